# baseline (speedup 1.0000x reference)
import jax
import jax.numpy as jnp
from jax import lax
from jax.experimental import pallas as pl
from jax.experimental.pallas import tpu as pltpu
from jax.experimental.pallas import tpu_sc as plsc

V = 1000000
D = 64
B = 16384
C = 20
NEG = 3
CN = C + NEG
NC = 2
NS = 16
NW = NC * NS
BPW = B // NW
CHUNK = 32
NSTEPS = BPW // CHUNK


def _sc_body(tidx_hbm, tpar_hbm, cidx_hbm, cpar_hbm,
             ctx_tbl_hbm, tgt_tbl_hbm, out_hbm,
             tidx_v, tpar_v, cidx_v, cpar_v,
             tgt_rows_v, ctx_rows_v, out_v, sem_c, sem_t):
    cid = lax.axis_index("c")
    sid = lax.axis_index("s")
    wid = sid * NC + cid
    base = wid * BPW

    def step(s, carry):
        ib = base + s * CHUNK
        pltpu.sync_copy(tidx_hbm.at[pl.ds(ib, CHUNK)], tidx_v)
        pltpu.sync_copy(tpar_hbm.at[pl.ds(ib, CHUNK)],
                        tpar_v.at[pl.ds(0, CHUNK)])
        pltpu.sync_copy(cidx_hbm.at[pl.ds(ib * CN, CHUNK * CN)], cidx_v)
        pltpu.sync_copy(cpar_hbm.at[pl.ds(ib * CN, CHUNK * CN)],
                        cpar_v.at[pl.ds(0, CHUNK * CN)])
        cp_t = pltpu.async_copy(tgt_tbl_hbm.at[tidx_v], tgt_rows_v, sem_t)
        cp_c = pltpu.async_copy(ctx_tbl_hbm.at[cidx_v], ctx_rows_v, sem_c)
        cp_t.wait()
        cp_c.wait()

        def item(i, carry2):
            ib23 = i * CN
            pt = tpar_v[pl.ds(i, 16)][0]
            t0 = tgt_rows_v[i, pl.ds(pt, 16)]
            t1 = tgt_rows_v[i, pl.ds(pt + 16, 16)]
            t2 = tgt_rows_v[i, pl.ds(pt + 32, 16)]
            t3 = tgt_rows_v[i, pl.ds(pt + 48, 16)]
            cs0 = jnp.zeros((16,), jnp.float32)
            cs1 = jnp.zeros((16,), jnp.float32)
            cs2 = jnp.zeros((16,), jnp.float32)
            cs3 = jnp.zeros((16,), jnp.float32)
            for j in range(C):
                pc = cpar_v[pl.ds(ib23 + j, 16)][0]
                cs0 = cs0 + ctx_rows_v[ib23 + j, pl.ds(pc, 16)]
                cs1 = cs1 + ctx_rows_v[ib23 + j, pl.ds(pc + 16, 16)]
                cs2 = cs2 + ctx_rows_v[ib23 + j, pl.ds(pc + 32, 16)]
                cs3 = cs3 + ctx_rows_v[ib23 + j, pl.ds(pc + 48, 16)]
            ns0 = jnp.zeros((16,), jnp.float32)
            ns1 = jnp.zeros((16,), jnp.float32)
            ns2 = jnp.zeros((16,), jnp.float32)
            ns3 = jnp.zeros((16,), jnp.float32)
            for j in range(C, CN):
                pn = cpar_v[pl.ds(ib23 + j, 16)][0]
                ns0 = ns0 + ctx_rows_v[ib23 + j, pl.ds(pn, 16)]
                ns1 = ns1 + ctx_rows_v[ib23 + j, pl.ds(pn + 16, 16)]
                ns2 = ns2 + ctx_rows_v[ib23 + j, pl.ds(pn + 32, 16)]
                ns3 = ns3 + ctx_rows_v[ib23 + j, pl.ds(pn + 48, 16)]
            pacc = cs0 * t0 + cs1 * t1 + cs2 * t2 + cs3 * t3
            nacc = ns0 * t0 + ns1 * t1 + ns2 * t2 + ns3 * t3
            out_v[i, pl.ds(0, 16)] = pacc
            out_v[i, pl.ds(16, 16)] = nacc
            return carry2

        lax.fori_loop(0, CHUNK, item, 0, unroll=False)
        pltpu.sync_copy(out_v, out_hbm.at[pl.ds(ib, CHUNK)])
        return carry

    lax.fori_loop(0, NSTEPS, step, 0, unroll=False)


def _tc_body(part_ref, out_ref):
    x = part_ref[...]
    p = jnp.sum(x[:, :16], axis=1) * (1.0 / C)
    n = -jnp.sum(x[:, 16:], axis=1)

    def logsig(v):
        return jnp.minimum(v, 0.0) - jnp.log1p(jnp.exp(-jnp.abs(v)))

    total = jnp.sum(logsig(p) + logsig(n))
    out_ref[0, 0] = -total * (1.0 / B)


def kernel(targets, contexts, negsamples, context_emb, target_emb):
    tidx = targets.astype(jnp.int32)
    cidx = jnp.concatenate(
        [contexts.astype(jnp.int32), negsamples.astype(jnp.int32)],
        axis=1).reshape(B * CN)
    tpacked, tpar = tidx >> 1, (tidx & 1) * 64
    cpacked, cpar = cidx >> 1, (cidx & 1) * 64
    ctx_tbl = context_emb.reshape(V // 2, 2 * D)
    tgt_tbl = target_emb.reshape(V // 2, 2 * D)

    mesh = plsc.VectorSubcoreMesh(core_axis_name="c", subcore_axis_name="s",
                                  num_cores=NC, num_subcores=NS)
    sc = pl.kernel(
        _sc_body,
        out_type=jax.ShapeDtypeStruct((B, 32), jnp.float32),
        mesh=mesh,
        scratch_types=[
            pltpu.VMEM((CHUNK,), jnp.int32),
            pltpu.VMEM((CHUNK + 16,), jnp.int32),
            pltpu.VMEM((CHUNK * CN,), jnp.int32),
            pltpu.VMEM((CHUNK * CN + 16,), jnp.int32),
            pltpu.VMEM((CHUNK, 2 * D), jnp.float32),
            pltpu.VMEM((CHUNK * CN, 2 * D), jnp.float32),
            pltpu.VMEM((CHUNK, 32), jnp.float32),
            pltpu.SemaphoreType.DMA,
            pltpu.SemaphoreType.DMA,
        ],
    )
    part = sc(tpacked, tpar, cpacked, cpar, ctx_tbl, tgt_tbl)

    loss = pl.pallas_call(
        _tc_body,
        out_shape=jax.ShapeDtypeStruct((1, 1), jnp.float32),
        in_specs=[pl.BlockSpec(memory_space=pltpu.VMEM)],
        out_specs=pl.BlockSpec(memory_space=pltpu.SMEM),
    )(part)
    return loss
